# Initial kernel scaffold; baseline (speedup 1.0000x reference)
import jax
import jax.numpy as jnp
from jax.experimental import pallas as pl


def kernel(feature, sp_edge_index, sp_value, edge_rel, ln1_g, ln1_b, ln2_g, ln2_b, Wq, bq, Wk, bk, Wv, bv, Wd, bd, W1, b1, W2, b2, rel_table, spatial_table):
    raise NotImplementedError("write your pallas kernel here")


# speedup vs baseline: 2.6661x; 2.6661x over previous
import jax
import jax.numpy as jnp
import numpy as np
from jax import lax
from jax.experimental import pallas as pl
from jax.experimental.pallas import tpu as pltpu
from jax.experimental.pallas import tpu_sc as plsc

N = 10000
E = 320000
D = 128
H = 8
DH = D // H
FF = 512
NUM_REL = 64
NUM_SP = 512
EPS = 1e-6
DN = float(H) ** -0.25
INV_D = np.float32(1.0 / D)

_NC = 2
_NS = 16
_NW = _NC * _NS
_C = 128
_NCHUNK = E // _C
_PW = -(-_NCHUNK // _NW)
_NCHUNK_CORE = E // 2 // _C
_PT = -(-_NCHUNK_CORE // _NS)
_NPAD = 10112
_RPT = _NPAD // _NS

_BN = 1000
_BE = 2000
_BN5 = 2000


def _row_sum(t):
    acc = t[:, 0:8]
    for i in range(1, 16):
        acc = acc + t[:, 8 * i:8 * i + 8]
    acc = acc[:, 0:4] + acc[:, 4:8]
    acc = acc[:, 0:2] + acc[:, 2:4]
    return acc[:, 0:1] + acc[:, 1:2]


def _ln(x, g, b):
    mu = _row_sum(x) * INV_D
    var = _row_sum((x - mu) ** 2) * INV_D
    return (x - mu) / jnp.sqrt(var + EPS) * g + b


def _head_sums_adj(p):
    t = p
    for s in (1, 2, 4, 8):
        t = t + jnp.roll(t, -s, axis=1)
    return jnp.concatenate([t[:, 16 * h:16 * h + 1] for h in range(H)], axis=1)


def _head_expand(a):
    return jnp.concatenate(
        [jnp.broadcast_to(a[:, h:h + 1], (a.shape[0], DH)) for h in range(H)],
        axis=1)


def _p1_body(feat_ref, g_ref, b_ref, wq_ref, bq_ref, wk_ref, bk_ref,
             wv_ref, bv_ref, q_ref, k_ref, v_ref, ks_ref, acc_ref):
    i = pl.program_id(0)
    x = _ln(feat_ref[...], g_ref[...], b_ref[...])
    q = jnp.dot(x, wq_ref[...], preferred_element_type=jnp.float32) + bq_ref[...]
    k = jnp.dot(x, wk_ref[...], preferred_element_type=jnp.float32) + bk_ref[...]
    v = jnp.dot(x, wv_ref[...], preferred_element_type=jnp.float32) + bv_ref[...]
    q_ref[...] = q
    k_ref[...] = k
    v_ref[...] = v

    @pl.when(i == 0)
    def _():
        acc_ref[...] = jnp.zeros_like(acc_ref)

    part = i // (N // 2 // _BN)

    def step(j, acc):
        return acc + k_ref[pl.ds(j * 8, 8), :]

    acc = lax.fori_loop(0, _BN // 8, step, acc_ref[pl.ds(part * 8, 8), :])
    acc_ref[pl.ds(part * 8, 8), :] = acc

    @pl.when(i == N // _BN - 1)
    def _():
        def fin(p):
            a = acc_ref[p * 8:(p + 1) * 8, :]
            a = a[0:4] + a[4:8]
            a = a[0:2] + a[2:4]
            return a[0:1] + a[1:2]
        ks_ref[...] = fin(0) + fin(1)


def _p1(feature, g, b, wq, bq, wk, bk, wv, bv):
    nblk = N // _BN
    row_spec = pl.BlockSpec((_BN, D), lambda i: (i, 0))
    full = lambda shape: pl.BlockSpec(shape, lambda i: tuple(0 for _ in shape))
    return pl.pallas_call(
        _p1_body,
        grid=(nblk,),
        in_specs=[row_spec, full((1, D)), full((1, D)), full((D, D)),
                  full((1, D)), full((D, D)), full((1, D)), full((D, D)),
                  full((1, D))],
        out_specs=[row_spec, row_spec, row_spec, full((1, D))],
        out_shape=[jax.ShapeDtypeStruct((N, D), jnp.float32)] * 3
        + [jax.ShapeDtypeStruct((1, D), jnp.float32)],
        scratch_shapes=[pltpu.VMEM((16, D), jnp.float32)],
    )(feature, g, b, wq, bq, wk, bk, wv, bv)


def _p2_body(q_hbm, k_hbm, v_hbm, row_hbm, col_hbm,
             qc_hbm, kc_hbm, vc_hbm, idx_v, buf_v, sem):
    wid = lax.axis_index("s") * _NC + lax.axis_index("c")

    def step(i, carry):
        ch = wid * _PW + i

        @pl.when(ch < _NCHUNK)
        def _():
            base = ch * _C
            pltpu.sync_copy(col_hbm.at[pl.ds(base, _C)], idx_v)
            pltpu.async_copy(q_hbm.at[idx_v], buf_v, sem).wait()
            pltpu.sync_copy(buf_v, qc_hbm.at[pl.ds(base, _C), :])
            pltpu.sync_copy(row_hbm.at[pl.ds(base, _C)], idx_v)
            pltpu.async_copy(k_hbm.at[idx_v], buf_v, sem).wait()
            pltpu.sync_copy(buf_v, kc_hbm.at[pl.ds(base, _C), :])
            pltpu.async_copy(v_hbm.at[idx_v], buf_v, sem).wait()
            pltpu.sync_copy(buf_v, vc_hbm.at[pl.ds(base, _C), :])

        return carry

    lax.fori_loop(0, _PW, step, 0)


def _p2(q, k, v, row, col):
    mesh = plsc.VectorSubcoreMesh(core_axis_name="c", subcore_axis_name="s")
    fn = pl.kernel(
        _p2_body,
        out_type=[jax.ShapeDtypeStruct((E, D), jnp.float32)] * 3,
        mesh=mesh,
        scratch_types=[
            pltpu.VMEM((_C,), jnp.int32),
            pltpu.VMEM((_C, D), jnp.float32),
            pltpu.SemaphoreType.DMA,
        ],
    )
    return fn(q, k, v, row, col)


def _p3_body(qc_ref, kc_ref, vc_ref, rel_ref, sp_ref, relt_ref, spt_ref,
             ks_ref, aw_ref, msg_ref):
    rel = rel_ref[0, 0, :]
    sp = sp_ref[0, 0, :]

    rel_oh = (
        rel[:, None] == lax.broadcasted_iota(jnp.int32, (_BE, NUM_REL), 1)
    ).astype(jnp.float32)
    re = jnp.dot(rel_oh, relt_ref[...], precision=lax.Precision.HIGHEST,
                 preferred_element_type=jnp.float32)

    qc = qc_ref[...]
    qe = qc + re
    ke = kc_ref[...] + re
    num = _head_sums_adj(qe * ke) * DN

    sp_oh = (
        sp[:, None] == lax.broadcasted_iota(jnp.int32, (_BE, NUM_SP), 1)
    ).astype(jnp.float32)
    bias = jnp.dot(sp_oh, spt_ref[...], precision=lax.Precision.HIGHEST,
                   preferred_element_type=jnp.float32)
    num = num + bias

    qb = qc.astype(jnp.bfloat16).astype(jnp.float32)
    ksb = ks_ref[...].astype(jnp.bfloat16).astype(jnp.float32)
    den = _head_sums_adj(qb * ksb)
    aw = num / den
    aw_ref[...] = aw
    msg_ref[...] = _head_expand(aw) * vc_ref[...]


def _p3(qc, kc, vc, rel3, sp3, rel_table, spatial_table, ks):
    nblk = E // _BE
    row_spec = pl.BlockSpec((_BE, D), lambda i: (i, 0))
    idx_spec = pl.BlockSpec((1, 1, _BE), lambda i: (i, 0, 0))
    full = lambda shape: pl.BlockSpec(shape, lambda i: tuple(0 for _ in shape))
    return pl.pallas_call(
        _p3_body,
        grid=(nblk,),
        in_specs=[row_spec, row_spec, row_spec, idx_spec, idx_spec,
                  full((NUM_REL, D)), full((NUM_SP, 1)), full((1, D))],
        out_specs=[pl.BlockSpec((_BE, H), lambda i: (i, 0)), row_spec],
        out_shape=[jax.ShapeDtypeStruct((E, H), jnp.float32),
                   jax.ShapeDtypeStruct((E, D), jnp.float32)],
    )(qc, kc, vc, rel3, sp3, rel_table, spatial_table, ks)


def _p4_body(msg_hbm, col_hbm, out_hbm, idx_v, buf_v, acc_sh, sem):
    cid = lax.axis_index("c")
    sid = lax.axis_index("s")
    base_row = sid * _RPT

    def zrow(r, carry):
        for j in range(D // 16):
            buf_v[r, pl.ds(j * 16, 16)] = jnp.zeros((16,), jnp.float32)
        return carry

    lax.fori_loop(0, _C, zrow, 0)

    nfull = _RPT // _C
    rem = _RPT - nfull * _C
    for j in range(nfull):
        pltpu.sync_copy(buf_v, acc_sh.at[pl.ds(base_row + j * _C, _C), :])
    pltpu.sync_copy(buf_v.at[pl.ds(0, rem), :],
                    acc_sh.at[pl.ds(base_row + nfull * _C, rem), :])

    plsc.subcore_barrier()

    def step(i, carry):
        ch = sid * _PT + i

        @pl.when(ch < _NCHUNK_CORE)
        def _():
            base = cid * (E // 2) + ch * _C
            pltpu.sync_copy(col_hbm.at[pl.ds(base, _C)], idx_v)
            pltpu.sync_copy(msg_hbm.at[pl.ds(base, _C), :], buf_v)
            pltpu.sync_copy(buf_v, acc_sh.at[idx_v], add=True)

        return carry

    lax.fori_loop(0, _PT, step, 0)

    plsc.subcore_barrier()

    for j in range(nfull):
        pltpu.sync_copy(acc_sh.at[pl.ds(base_row + j * _C, _C), :],
                        out_hbm.at[cid, pl.ds(base_row + j * _C, _C), :])
    pltpu.sync_copy(acc_sh.at[pl.ds(base_row + nfull * _C, rem), :],
                    out_hbm.at[cid, pl.ds(base_row + nfull * _C, rem), :])


def _p4(msgs, col):
    mesh = plsc.VectorSubcoreMesh(core_axis_name="c", subcore_axis_name="s")
    fn = pl.kernel(
        _p4_body,
        out_type=jax.ShapeDtypeStruct((2, _NPAD, D), jnp.float32),
        mesh=mesh,
        scratch_types=[
            pltpu.VMEM((_C,), jnp.int32),
            pltpu.VMEM((_C, D), jnp.float32),
            pltpu.VMEM_SHARED((_NPAD, D), jnp.float32),
            pltpu.SemaphoreType.DMA,
        ],
    )
    return fn(msgs, col)


def _p5_body(part_ref, feat_ref, wd_ref, bd_ref, g_ref, b_ref,
             w1_ref, b1_ref, w2_ref, b2_ref, out_ref):
    agg = part_ref[0] + part_ref[1]
    attn = jnp.dot(agg, wd_ref[...], preferred_element_type=jnp.float32) + bd_ref[...]
    out1 = attn + feat_ref[...]
    out1n = _ln(out1, g_ref[...], b_ref[...])
    h = jnp.maximum(
        jnp.dot(out1n, w1_ref[...], preferred_element_type=jnp.float32) + b1_ref[...],
        0.0)
    out_ref[...] = out1 + jnp.dot(
        h, w2_ref[...], preferred_element_type=jnp.float32) + b2_ref[...]


def _p5(part, feature, wd, bd, g, b, w1, b1, w2, b2):
    nblk = N // _BN5
    row_spec = pl.BlockSpec((_BN5, D), lambda i: (i, 0))
    full = lambda shape: pl.BlockSpec(shape, lambda i: tuple(0 for _ in shape))
    return pl.pallas_call(
        _p5_body,
        grid=(nblk,),
        in_specs=[pl.BlockSpec((2, _BN5, D), lambda i: (0, i, 0)), row_spec,
                  full((D, D)), full((1, D)), full((1, D)), full((1, D)),
                  full((D, FF)), full((1, FF)), full((FF, D)), full((1, D))],
        out_specs=row_spec,
        out_shape=jax.ShapeDtypeStruct((N, D), jnp.float32),
    )(part, feature, wd, bd, g, b, w1, b1, w2, b2)


def kernel(feature, sp_edge_index, sp_value, edge_rel, ln1_g, ln1_b, ln2_g,
           ln2_b, Wq, bq, Wk, bk, Wv, bv, Wd, bd, W1, b1, W2, b2, rel_table,
           spatial_table):
    r2 = lambda t: t.reshape(1, -1)
    q, k, v, ks = _p1(feature, r2(ln1_g), r2(ln1_b), Wq, r2(bq), Wk, r2(bk),
                      Wv, r2(bv))
    row = sp_edge_index[0]
    col = sp_edge_index[1]
    qc, kc, vc = _p2(q, k, v, row, col)
    rel3 = edge_rel.reshape(E // _BE, 1, _BE)
    sp3 = sp_value.reshape(E // _BE, 1, _BE)
    aw, msgs = _p3(qc, kc, vc, rel3, sp3, rel_table, spatial_table, ks)
    part = _p4(msgs, col)
    out2 = _p5(part, feature, Wd, r2(bd), r2(ln2_g), r2(ln2_b), W1, r2(b1),
               W2, r2(b2))
    return (out2, aw)


# speedup vs baseline: 3.0502x; 1.1441x over previous
import jax
import jax.numpy as jnp
import numpy as np
from jax import lax
from jax.experimental import pallas as pl
from jax.experimental.pallas import tpu as pltpu
from jax.experimental.pallas import tpu_sc as plsc

N = 10000
E = 320000
D = 128
H = 8
DH = D // H
FF = 512
NUM_REL = 64
NUM_SP = 512
EPS = 1e-6
DN = float(H) ** -0.25
INV_D = np.float32(1.0 / D)

_NC = 2
_NS = 16
_NW = _NC * _NS
_C = 128
_NCHUNK = E // _C
_PW = -(-_NCHUNK // _NW)
_NCHUNK_CORE = E // 2 // _C
_PT = -(-_NCHUNK_CORE // _NS)
_NPAD = 10112
_RPT = _NPAD // _NS

_BN = 1000
_BE = 2000
_BN5 = 2000


def _row_sum(t):
    acc = t[:, 0:8]
    for i in range(1, 16):
        acc = acc + t[:, 8 * i:8 * i + 8]
    acc = acc[:, 0:4] + acc[:, 4:8]
    acc = acc[:, 0:2] + acc[:, 2:4]
    return acc[:, 0:1] + acc[:, 1:2]


def _ln(x, g, b):
    mu = _row_sum(x) * INV_D
    var = _row_sum((x - mu) ** 2) * INV_D
    return (x - mu) / jnp.sqrt(var + EPS) * g + b


def _head_sums_adj(p):
    t = p
    for s in (1, 2, 4, 8):
        t = t + jnp.roll(t, -s, axis=1)
    return jnp.concatenate([t[:, 16 * h:16 * h + 1] for h in range(H)], axis=1)


def _head_expand(a):
    return jnp.concatenate(
        [jnp.broadcast_to(a[:, h:h + 1], (a.shape[0], DH)) for h in range(H)],
        axis=1)


def _p1_body(feat_ref, g_ref, b_ref, wq_ref, bq_ref, wk_ref, bk_ref,
             wv_ref, bv_ref, q_ref, k_ref, v_ref, ks_ref, acc_ref):
    i = pl.program_id(0)
    x = _ln(feat_ref[...], g_ref[...], b_ref[...])
    q = jnp.dot(x, wq_ref[...], preferred_element_type=jnp.float32) + bq_ref[...]
    k = jnp.dot(x, wk_ref[...], preferred_element_type=jnp.float32) + bk_ref[...]
    v = jnp.dot(x, wv_ref[...], preferred_element_type=jnp.float32) + bv_ref[...]
    q_ref[...] = q
    k_ref[...] = k
    v_ref[...] = v

    @pl.when(i == 0)
    def _():
        acc_ref[...] = jnp.zeros_like(acc_ref)

    part = i // (N // 2 // _BN)

    def step(j, acc):
        return acc + k_ref[pl.ds(j * 8, 8), :]

    acc = lax.fori_loop(0, _BN // 8, step, acc_ref[pl.ds(part * 8, 8), :])
    acc_ref[pl.ds(part * 8, 8), :] = acc

    @pl.when(i == N // _BN - 1)
    def _():
        def fin(p):
            a = acc_ref[p * 8:(p + 1) * 8, :]
            a = a[0:4] + a[4:8]
            a = a[0:2] + a[2:4]
            return a[0:1] + a[1:2]
        ks_ref[...] = fin(0) + fin(1)


def _p1(feature, g, b, wq, bq, wk, bk, wv, bv):
    nblk = N // _BN
    row_spec = pl.BlockSpec((_BN, D), lambda i: (i, 0))
    full = lambda shape: pl.BlockSpec(shape, lambda i: tuple(0 for _ in shape))
    return pl.pallas_call(
        _p1_body,
        grid=(nblk,),
        in_specs=[row_spec, full((1, D)), full((1, D)), full((D, D)),
                  full((1, D)), full((D, D)), full((1, D)), full((D, D)),
                  full((1, D))],
        out_specs=[row_spec, row_spec, row_spec, full((1, D))],
        out_shape=[jax.ShapeDtypeStruct((N, D), jnp.float32)] * 3
        + [jax.ShapeDtypeStruct((1, D), jnp.float32)],
        scratch_shapes=[pltpu.VMEM((16, D), jnp.float32)],
    )(feature, g, b, wq, bq, wk, bk, wv, bv)


def _p2_body(nchunk, pw, q_hbm, k_hbm, v_hbm, row_hbm, col_hbm,
             qc_hbm, kc_hbm, vc_hbm,
             idxc0, idxr0, idxc1, idxr1,
             bq0, bk0, bv0, bq1, bk1, bv1, sem_g, sem_w):
    wid = lax.axis_index("s") * _NC + lax.axis_index("c")
    sets = [(idxc0, idxr0, bq0, bk0, bv0), (idxc1, idxr1, bq1, bk1, bv1)]

    def step(j, carry):
        chs = [wid * pw + 2 * j, wid * pw + 2 * j + 1]
        valid = [(2 * j < pw) & (chs[0] < nchunk),
                 (2 * j + 1 < pw) & (chs[1] < nchunk)]
        for s in range(2):
            ic, ir, bq, bk, bv = sets[s]

            @pl.when(valid[s])
            def _(s=s, ic=ic, ir=ir, bq=bq, bk=bk, bv=bv):
                base = chs[s] * _C
                pltpu.sync_copy(col_hbm.at[pl.ds(base, _C)], ic)
                pltpu.sync_copy(row_hbm.at[pl.ds(base, _C)], ir)
                pltpu.async_copy(q_hbm.at[ic], bq, sem_g)
                pltpu.async_copy(k_hbm.at[ir], bk, sem_g)
                pltpu.async_copy(v_hbm.at[ir], bv, sem_g)

        for s in range(2):
            ic, ir, bq, bk, bv = sets[s]

            @pl.when(valid[s])
            def _(s=s, ic=ic, ir=ir, bq=bq, bk=bk, bv=bv):
                base = chs[s] * _C
                pltpu.make_async_copy(q_hbm.at[ic], bq, sem_g).wait()
                pltpu.make_async_copy(k_hbm.at[ir], bk, sem_g).wait()
                pltpu.make_async_copy(v_hbm.at[ir], bv, sem_g).wait()
                pltpu.async_copy(bq, qc_hbm.at[pl.ds(base, _C), :], sem_w)
                pltpu.async_copy(bk, kc_hbm.at[pl.ds(base, _C), :], sem_w)
                pltpu.async_copy(bv, vc_hbm.at[pl.ds(base, _C), :], sem_w)

        for s in range(2):
            ic, ir, bq, bk, bv = sets[s]

            @pl.when(valid[s])
            def _(s=s, ic=ic, ir=ir, bq=bq, bk=bk, bv=bv):
                base = chs[s] * _C
                pltpu.make_async_copy(bq, qc_hbm.at[pl.ds(base, _C), :], sem_w).wait()
                pltpu.make_async_copy(bk, kc_hbm.at[pl.ds(base, _C), :], sem_w).wait()
                pltpu.make_async_copy(bv, vc_hbm.at[pl.ds(base, _C), :], sem_w).wait()

        return carry

    lax.fori_loop(0, (pw + 1) // 2, step, 0)


def _p2(q, k, v, row, col, e):
    nchunk = e // _C
    pw = -(-nchunk // _NW)
    mesh = plsc.VectorSubcoreMesh(core_axis_name="c", subcore_axis_name="s")
    fn = pl.kernel(
        lambda *refs: _p2_body(nchunk, pw, *refs),
        out_type=[jax.ShapeDtypeStruct((e, D), jnp.float32)] * 3,
        mesh=mesh,
        scratch_types=[
            pltpu.VMEM((_C,), jnp.int32),
            pltpu.VMEM((_C,), jnp.int32),
            pltpu.VMEM((_C,), jnp.int32),
            pltpu.VMEM((_C,), jnp.int32),
            pltpu.VMEM((_C, D), jnp.float32),
            pltpu.VMEM((_C, D), jnp.float32),
            pltpu.VMEM((_C, D), jnp.float32),
            pltpu.VMEM((_C, D), jnp.float32),
            pltpu.VMEM((_C, D), jnp.float32),
            pltpu.VMEM((_C, D), jnp.float32),
            pltpu.SemaphoreType.DMA,
            pltpu.SemaphoreType.DMA,
        ],
    )
    return fn(q, k, v, row, col)


def _p3_body(qc_ref, kc_ref, vc_ref, rel_ref, sp_ref, relt_ref, spt_ref,
             ks_ref, aw_ref, msg_ref):
    rel = rel_ref[0, 0, :]
    sp = sp_ref[0, 0, :]

    rel_oh = (
        rel[:, None] == lax.broadcasted_iota(jnp.int32, (_BE, NUM_REL), 1)
    ).astype(jnp.float32)
    re = jnp.dot(rel_oh, relt_ref[...], precision=lax.Precision.HIGHEST,
                 preferred_element_type=jnp.float32)

    qc = qc_ref[...]
    qe = qc + re
    ke = kc_ref[...] + re
    num = _head_sums_adj(qe * ke) * DN

    sp_oh = (
        sp[:, None] == lax.broadcasted_iota(jnp.int32, (_BE, NUM_SP), 1)
    ).astype(jnp.float32)
    bias = jnp.dot(sp_oh, spt_ref[...], precision=lax.Precision.HIGHEST,
                   preferred_element_type=jnp.float32)
    num = num + bias

    qb = qc.astype(jnp.bfloat16).astype(jnp.float32)
    ksb = ks_ref[...].astype(jnp.bfloat16).astype(jnp.float32)
    den = _head_sums_adj(qb * ksb)
    aw = num / den
    aw_ref[...] = aw
    msg_ref[...] = _head_expand(aw) * vc_ref[...]


def _p3(qc, kc, vc, rel3, sp3, rel_table, spatial_table, ks):
    e = qc.shape[0]
    nblk = e // _BE
    row_spec = pl.BlockSpec((_BE, D), lambda i: (i, 0))
    idx_spec = pl.BlockSpec((1, 1, _BE), lambda i: (i, 0, 0))
    full = lambda shape: pl.BlockSpec(shape, lambda i: tuple(0 for _ in shape))
    return pl.pallas_call(
        _p3_body,
        grid=(nblk,),
        in_specs=[row_spec, row_spec, row_spec, idx_spec, idx_spec,
                  full((NUM_REL, D)), full((NUM_SP, 1)), full((1, D))],
        out_specs=[pl.BlockSpec((_BE, H), lambda i: (i, 0)), row_spec],
        out_shape=[jax.ShapeDtypeStruct((e, H), jnp.float32),
                   jax.ShapeDtypeStruct((e, D), jnp.float32)],
    )(qc, kc, vc, rel3, sp3, rel_table, spatial_table, ks)


def _p4_body(msg0_hbm, msg1_hbm, col_hbm, out_hbm, idx_v, buf_v, acc_sh, sem):
    cid = lax.axis_index("c")
    sid = lax.axis_index("s")
    base_row = sid * _RPT

    def zrow(r, carry):
        for j in range(D // 16):
            buf_v[r, pl.ds(j * 16, 16)] = jnp.zeros((16,), jnp.float32)
        return carry

    lax.fori_loop(0, _C, zrow, 0)

    nfull = _RPT // _C
    rem = _RPT - nfull * _C
    for j in range(nfull):
        pltpu.sync_copy(buf_v, acc_sh.at[pl.ds(base_row + j * _C, _C), :])
    pltpu.sync_copy(buf_v.at[pl.ds(0, rem), :],
                    acc_sh.at[pl.ds(base_row + nfull * _C, rem), :])

    plsc.subcore_barrier()

    def step(i, carry):
        ch = sid * _PT + i

        @pl.when(ch < _NCHUNK_CORE)
        def _():
            base = cid * (E // 2) + ch * _C
            pltpu.sync_copy(col_hbm.at[pl.ds(base, _C)], idx_v)

            @pl.when(cid == 0)
            def _():
                pltpu.sync_copy(msg0_hbm.at[pl.ds(ch * _C, _C), :], buf_v)

            @pl.when(cid == 1)
            def _():
                pltpu.sync_copy(msg1_hbm.at[pl.ds(ch * _C, _C), :], buf_v)

            pltpu.sync_copy(buf_v, acc_sh.at[idx_v], add=True)

        return carry

    lax.fori_loop(0, _PT, step, 0)

    plsc.subcore_barrier()

    for j in range(nfull):
        pltpu.sync_copy(acc_sh.at[pl.ds(base_row + j * _C, _C), :],
                        out_hbm.at[cid, pl.ds(base_row + j * _C, _C), :])
    pltpu.sync_copy(acc_sh.at[pl.ds(base_row + nfull * _C, rem), :],
                    out_hbm.at[cid, pl.ds(base_row + nfull * _C, rem), :])


def _p4(msgs0, msgs1, col):
    mesh = plsc.VectorSubcoreMesh(core_axis_name="c", subcore_axis_name="s")
    fn = pl.kernel(
        _p4_body,
        out_type=jax.ShapeDtypeStruct((2, _NPAD, D), jnp.float32),
        mesh=mesh,
        scratch_types=[
            pltpu.VMEM((_C,), jnp.int32),
            pltpu.VMEM((_C, D), jnp.float32),
            pltpu.VMEM_SHARED((_NPAD, D), jnp.float32),
            pltpu.SemaphoreType.DMA,
        ],
    )
    return fn(msgs0, msgs1, col)


def _p5_body(part_ref, feat_ref, wd_ref, bd_ref, g_ref, b_ref,
             w1_ref, b1_ref, w2_ref, b2_ref, out_ref):
    agg = part_ref[0] + part_ref[1]
    attn = jnp.dot(agg, wd_ref[...], preferred_element_type=jnp.float32) + bd_ref[...]
    out1 = attn + feat_ref[...]
    out1n = _ln(out1, g_ref[...], b_ref[...])
    h = jnp.maximum(
        jnp.dot(out1n, w1_ref[...], preferred_element_type=jnp.float32) + b1_ref[...],
        0.0)
    out_ref[...] = out1 + jnp.dot(
        h, w2_ref[...], preferred_element_type=jnp.float32) + b2_ref[...]


def _p5(part, feature, wd, bd, g, b, w1, b1, w2, b2):
    nblk = N // _BN5
    row_spec = pl.BlockSpec((_BN5, D), lambda i: (i, 0))
    full = lambda shape: pl.BlockSpec(shape, lambda i: tuple(0 for _ in shape))
    return pl.pallas_call(
        _p5_body,
        grid=(nblk,),
        in_specs=[pl.BlockSpec((2, _BN5, D), lambda i: (0, i, 0)), row_spec,
                  full((D, D)), full((1, D)), full((1, D)), full((1, D)),
                  full((D, FF)), full((1, FF)), full((FF, D)), full((1, D))],
        out_specs=row_spec,
        out_shape=jax.ShapeDtypeStruct((N, D), jnp.float32),
    )(part, feature, wd, bd, g, b, w1, b1, w2, b2)


def kernel(feature, sp_edge_index, sp_value, edge_rel, ln1_g, ln1_b, ln2_g,
           ln2_b, Wq, bq, Wk, bk, Wv, bv, Wd, bd, W1, b1, W2, b2, rel_table,
           spatial_table):
    r2 = lambda t: t.reshape(1, -1)
    q, k, v, ks = _p1(feature, r2(ln1_g), r2(ln1_b), Wq, r2(bq), Wk, r2(bk),
                      Wv, r2(bv))
    row = sp_edge_index[0]
    col = sp_edge_index[1]
    e2 = E // 2
    halves = []
    for h in range(2):
        sl = slice(h * e2, (h + 1) * e2)
        qc, kc, vc = _p2(q, k, v, row[sl], col[sl], e2)
        rel3 = edge_rel[sl].reshape(e2 // _BE, 1, _BE)
        sp3 = sp_value[sl].reshape(e2 // _BE, 1, _BE)
        halves.append(_p3(qc, kc, vc, rel3, sp3, rel_table, spatial_table, ks))
    aw = jnp.concatenate([halves[0][0], halves[1][0]], axis=0)
    part = _p4(halves[0][1], halves[1][1], col)
    out2 = _p5(part, feature, Wd, r2(bd), r2(ln2_g), r2(ln2_b), W1, r2(b1),
               W2, r2(b2))
    return (out2, aw)
